# Initial kernel scaffold; baseline (speedup 1.0000x reference)
import jax
import jax.numpy as jnp
from jax.experimental import pallas as pl


def kernel(x, edge_index, W1, b1, W2, b2):
    raise NotImplementedError("write your pallas kernel here")


# speedup vs baseline: 12.7695x; 12.7695x over previous
import functools

import jax
import jax.numpy as jnp
from jax import lax
from jax.experimental import pallas as pl
from jax.experimental.pallas import tpu as pltpu
from jax.experimental.pallas import tpu_sc as plsc

N = 10000
D = 128
H = 1024
E = 320000

NC = 2
NS = 16
NW = NC * NS
CH = 128
CPT = 80
EPT = CPT * CH
EP = NW * EPT
NP = 10240
NPT = NP // NS
RB = 1024


@functools.cache
def _build_deg_kernel():
    mesh = plsc.VectorSubcoreMesh(
        core_axis_name="c", subcore_axis_name="s",
        num_cores=NC, num_subcores=NS)
    return pl.kernel(
        _deg_body,
        out_type=jax.ShapeDtypeStruct((NC, NP, 16), jnp.float32),
        mesh=mesh,
        scratch_types=[
            pltpu.VMEM((CH,), jnp.int32),
            pltpu.VMEM((CH, 16), jnp.float32),
            pltpu.VMEM_SHARED((NP, 16), jnp.float32),
        ],
    )


def _deg_body(dst_hbm, z16_hbm, o16_hbm, out_hbm, idx_v, ones_v, acc):
    c = lax.axis_index("c")
    s = lax.axis_index("s")
    wid = c * NS + s
    pltpu.sync_copy(z16_hbm, acc.at[pl.ds(s * NPT, NPT)])
    pltpu.sync_copy(o16_hbm, ones_v)
    plsc.subcore_barrier()
    base = wid * EPT

    def body(j, carry):
        off = pl.multiple_of(base + j * CH, CH)
        pltpu.sync_copy(dst_hbm.at[pl.ds(off, CH)], idx_v)
        pltpu.sync_copy(ones_v, acc.at[idx_v], add=True)
        return carry

    lax.fori_loop(0, CPT, body, 0)
    plsc.subcore_barrier()
    pltpu.sync_copy(acc.at[pl.ds(s * NPT, NPT)],
                    out_hbm.at[c, pl.ds(s * NPT, NPT)])


@functools.cache
def _build_scatter_kernel():
    mesh = plsc.VectorSubcoreMesh(
        core_axis_name="c", subcore_axis_name="s",
        num_cores=NC, num_subcores=NS)
    return pl.kernel(
        _scatter_body,
        out_type=jax.ShapeDtypeStruct((NC, NP, D), jnp.float32),
        mesh=mesh,
        scratch_types=[
            pltpu.VMEM((2, CH), jnp.int32),
            pltpu.VMEM((2, CH), jnp.int32),
            pltpu.VMEM((CH, D), jnp.float32),
            pltpu.VMEM((CH, D), jnp.float32),
            pltpu.VMEM_SHARED((NP, D), jnp.float32),
            pltpu.SemaphoreType.DMA,
            pltpu.SemaphoreType.DMA,
        ],
    )


def _scatter_body(tbl_hbm, src_hbm, dst_hbm, z_hbm, out_hbm,
                  isrc, idst, rowa, rowb, acc, sema, semb):
    c = lax.axis_index("c")
    s = lax.axis_index("s")
    wid = c * NS + s
    pltpu.sync_copy(z_hbm, acc.at[pl.ds(s * NPT, NPT)])
    plsc.subcore_barrier()
    base = wid * EPT

    pltpu.sync_copy(src_hbm.at[pl.ds(pl.multiple_of(base, CH), CH)],
                    isrc.at[0])
    pltpu.async_copy(tbl_hbm.at[isrc.at[0]], rowa, sema)

    def body(g, carry):
        j0 = pl.multiple_of(base + (2 * g) * CH, CH)
        j1 = pl.multiple_of(base + (2 * g + 1) * CH, CH)
        j2 = pl.multiple_of(base + (2 * g + 2) * CH, CH)
        pltpu.sync_copy(src_hbm.at[pl.ds(j1, CH)], isrc.at[1])
        pltpu.async_copy(tbl_hbm.at[isrc.at[1]], rowb, semb)
        pltpu.make_async_copy(tbl_hbm.at[isrc.at[0]], rowa, sema).wait()
        pltpu.sync_copy(dst_hbm.at[pl.ds(j0, CH)], idst.at[0])
        pltpu.sync_copy(rowa, acc.at[idst.at[0]], add=True)
        pltpu.sync_copy(src_hbm.at[pl.ds(j2, CH)], isrc.at[0])
        pltpu.async_copy(tbl_hbm.at[isrc.at[0]], rowa, sema)
        pltpu.make_async_copy(tbl_hbm.at[isrc.at[1]], rowb, semb).wait()
        pltpu.sync_copy(dst_hbm.at[pl.ds(j1, CH)], idst.at[1])
        pltpu.sync_copy(rowb, acc.at[idst.at[1]], add=True)
        return carry

    lax.fori_loop(0, CPT // 2, body, 0)
    pltpu.make_async_copy(tbl_hbm.at[isrc.at[0]], rowa, sema).wait()
    plsc.subcore_barrier()
    pltpu.sync_copy(acc.at[pl.ds(s * NPT, NPT)],
                    out_hbm.at[c, pl.ds(s * NPT, NPT)])


def _prep_body(degp_ref, x_ref, dinv_ref, xs_ref):
    dp = degp_ref[...]
    deg = dp[0, :, 0:1] + dp[1, :, 0:1] + 1.0
    dinv = lax.rsqrt(deg)
    dinv_ref[...] = jnp.broadcast_to(dinv, (RB, D))
    xs_ref[...] = x_ref[...] * dinv


def _mlp_body(s1_ref, xs_ref, dinv_ref, w1_ref, b1_ref, w2_ref, zs_ref):
    dinv = dinv_ref[...]
    a1 = dinv * (s1_ref[0] + s1_ref[1] + xs_ref[...])
    h = jnp.dot(a1, w1_ref[...], preferred_element_type=jnp.float32)
    h = jnp.maximum(h + b1_ref[...], 0.0)
    z = jnp.dot(h, w2_ref[...], preferred_element_type=jnp.float32)
    zs_ref[...] = dinv * z


def _out_body(s2_ref, zs_ref, dinv_ref, b2_ref, o_ref):
    v = dinv_ref[...] * (s2_ref[0] + s2_ref[1] + zs_ref[...]) + b2_ref[...]
    o_ref[...] = v.T


def kernel(x, edge_index, W1, b1, W2, b2):
    src = edge_index[0]
    dst = edge_index[1]
    pad = EP - E
    srcp = jnp.concatenate(
        [src, jnp.full((pad + CH,), N, jnp.int32)])
    dstp = jnp.concatenate(
        [dst, jnp.full((pad + CH,), N, jnp.int32)])
    xp = jnp.concatenate([x, jnp.zeros((NP - N, D), x.dtype)])
    z640 = jnp.zeros((NPT, D), jnp.float32)
    z16 = jnp.zeros((NPT, 16), jnp.float32)
    o16 = jnp.ones((CH, 16), jnp.float32)
    b1r = b1.reshape(1, H)
    b2r = b2.reshape(1, D)

    degp = _build_deg_kernel()(dstp, z16, o16)

    dinv_bc, xs = pl.pallas_call(
        _prep_body,
        grid=(NP // RB,),
        in_specs=[
            pl.BlockSpec((NC, RB, 16), lambda i: (0, i, 0)),
            pl.BlockSpec((RB, D), lambda i: (i, 0)),
        ],
        out_specs=[
            pl.BlockSpec((RB, D), lambda i: (i, 0)),
            pl.BlockSpec((RB, D), lambda i: (i, 0)),
        ],
        out_shape=[
            jax.ShapeDtypeStruct((NP, D), jnp.float32),
            jax.ShapeDtypeStruct((NP, D), jnp.float32),
        ],
    )(degp, xp)

    s1p = _build_scatter_kernel()(xs, srcp, dstp, z640)

    zs = pl.pallas_call(
        _mlp_body,
        grid=(NP // RB,),
        in_specs=[
            pl.BlockSpec((NC, RB, D), lambda i: (0, i, 0)),
            pl.BlockSpec((RB, D), lambda i: (i, 0)),
            pl.BlockSpec((RB, D), lambda i: (i, 0)),
            pl.BlockSpec((D, H), lambda i: (0, 0)),
            pl.BlockSpec((1, H), lambda i: (0, 0)),
            pl.BlockSpec((H, D), lambda i: (0, 0)),
        ],
        out_specs=pl.BlockSpec((RB, D), lambda i: (i, 0)),
        out_shape=jax.ShapeDtypeStruct((NP, D), jnp.float32),
    )(s1p, xs, dinv_bc, W1, b1r, W2)

    s2p = _build_scatter_kernel()(zs, srcp, dstp, z640)

    out = pl.pallas_call(
        _out_body,
        grid=(NP // RB,),
        in_specs=[
            pl.BlockSpec((NC, RB, D), lambda i: (0, i, 0)),
            pl.BlockSpec((RB, D), lambda i: (i, 0)),
            pl.BlockSpec((RB, D), lambda i: (i, 0)),
            pl.BlockSpec((1, D), lambda i: (0, 0)),
        ],
        out_specs=pl.BlockSpec((D, RB), lambda i: (0, i)),
        out_shape=jax.ShapeDtypeStruct((D, N), jnp.float32),
    )(s2p, zs, dinv_bc, b2r)

    return out
